# Initial kernel scaffold; baseline (speedup 1.0000x reference)
import jax
import jax.numpy as jnp
from jax.experimental import pallas as pl


def kernel(x, attention_mask, wp_w, wp_b, kp_w1, kp_b1, kp_w2, kp_b2, ln1_g, ln1_b, w_qkv, b_qkv, w_o, b_o, ln2_g, ln2_b, w_ff1, b_ff1, w_ff2, b_ff2):
    raise NotImplementedError("write your pallas kernel here")


# speedup vs baseline: 2.6710x; 2.6710x over previous
import functools

import jax
import jax.numpy as jnp
import numpy as np
from jax.experimental import pallas as pl
from jax.experimental.pallas import tpu as pltpu

B, S, D = 8, 512, 256
MAX_TOKENS = 512
H, DH, DFF = 8, 32, 1024
K_TOTAL = S * D

ROUTER_CHUNK = 4096
ROUTER_STEPS = K_TOTAL // ROUTER_CHUNK


def _router_kernel(xf_ref, w1_ref, b1_ref, w2_ref, kl_ref, acc_ref):
    i = pl.program_id(0)

    @pl.when(i == 0)
    def _init():
        acc_ref[...] = jnp.zeros_like(acc_ref)

    acc_ref[...] += jnp.dot(
        xf_ref[...].astype(jnp.bfloat16),
        w1_ref[...].astype(jnp.bfloat16),
        preferred_element_type=jnp.float32,
    )

    @pl.when(i == ROUTER_STEPS - 1)
    def _epilogue():
        hdn = acc_ref[...] + b1_ref[...]
        hdn = jnp.where(hdn >= 0, hdn, 0.01 * hdn)
        kl_ref[...] = jnp.sum(hdn * w2_ref[...], axis=1, keepdims=True)


def _block_kernel(thr_ref, x_ref, mask_ref, wp_ref, wpb_ref,
                  ln1g_ref, ln1b_ref, wqkv_ref, bqkv_ref, wo_ref, bo_ref,
                  ln2g_ref, ln2b_ref, wff1_ref, bff1_ref, wff2_ref, bff2_ref,
                  out_ref):
    b = pl.program_id(0)
    xs = x_ref[0]

    weights = jnp.sum(xs * wp_ref[...], axis=1) + wpb_ref[0, 0]
    thr_f = thr_ref[b].astype(jnp.float32)
    sel = weights > thr_f
    any_sel = jnp.any(sel)

    @pl.when(jnp.logical_not(any_sel))
    def _copy():
        out_ref[0] = xs

    @pl.when(any_sel)
    def _block():
        def ln(v, g, bb):
            mu = jnp.mean(v, axis=1, keepdims=True)
            var = jnp.mean((v - mu) ** 2, axis=1, keepdims=True)
            return (v - mu) / jnp.sqrt(var + 1e-5) * g + bb

        def mm(a, w):
            return jax.lax.dot_general(
                a.astype(jnp.bfloat16), w.astype(jnp.bfloat16),
                (((1,), (0,)), ((), ())),
                preferred_element_type=jnp.float32,
            )

        a = ln(xs, ln1g_ref[...], ln1b_ref[...])
        qkv = mm(a, wqkv_ref[...]) + bqkv_ref[...]
        bias = mask_ref[0, 0] + jnp.where(sel, 0.0, -1e9)

        ctx_parts = []
        for h in range(H):
            q = qkv[:, h * DH:(h + 1) * DH]
            k = qkv[:, D + h * DH:D + (h + 1) * DH]
            v = qkv[:, 2 * D + h * DH:2 * D + (h + 1) * DH]
            s = jax.lax.dot_general(
                q.astype(jnp.bfloat16), k.astype(jnp.bfloat16),
                (((1,), (1,)), ((), ())),
                preferred_element_type=jnp.float32,
            ) * (1.0 / np.sqrt(DH)) + bias[None, :]
            m = jnp.max(s, axis=1, keepdims=True)
            p = jnp.exp(s - m)
            p = p / jnp.sum(p, axis=1, keepdims=True)
            ctx_parts.append(mm(p, v))
        ctx = jnp.concatenate(ctx_parts, axis=1)

        h1 = xs + mm(ctx, wo_ref[...]) + bo_ref[...]
        m2 = ln(h1, ln2g_ref[...], ln2b_ref[...])
        ff = jax.nn.gelu(mm(m2, wff1_ref[...]) + bff1_ref[...])
        blk = h1 + mm(ff, wff2_ref[...]) + bff2_ref[...]

        selw = jnp.where(sel, weights, 0.0)
        out_ref[0] = xs + selw[:, None] * blk


def kernel(x, attention_mask, wp_w, wp_b, kp_w1, kp_b1, kp_w2, kp_b2,
           ln1_g, ln1_b, w_qkv, b_qkv, w_o, b_o, ln2_g, ln2_b,
           w_ff1, b_ff1, w_ff2, b_ff2):
    x_flat = x.reshape(B, K_TOTAL)

    k_logits = pl.pallas_call(
        _router_kernel,
        grid=(ROUTER_STEPS,),
        in_specs=[
            pl.BlockSpec((B, ROUTER_CHUNK), lambda i: (0, i)),
            pl.BlockSpec((ROUTER_CHUNK, 512), lambda i: (i, 0)),
            pl.BlockSpec((1, 512), lambda i: (0, 0)),
            pl.BlockSpec((1, 512), lambda i: (0, 0)),
        ],
        out_specs=pl.BlockSpec((B, 1), lambda i: (0, 0)),
        out_shape=jax.ShapeDtypeStruct((B, 1), jnp.float32),
        scratch_shapes=[pltpu.VMEM((B, 512), jnp.float32)],
    )(x_flat, kp_w1, kp_b1.reshape(1, 512), kp_w2.reshape(1, 512))

    thr = jnp.clip(
        jax.nn.sigmoid(k_logits[:, 0] + kp_b2[0]) * MAX_TOKENS, 1, MAX_TOKENS
    ).astype(jnp.int32)

    row = lambda v: v.reshape(1, -1)
    const = lambda shape: pl.BlockSpec(shape, lambda b, thr_ref: tuple(0 for _ in shape))

    out = pl.pallas_call(
        _block_kernel,
        grid_spec=pltpu.PrefetchScalarGridSpec(
            num_scalar_prefetch=1,
            grid=(B,),
            in_specs=[
                pl.BlockSpec((1, S, D), lambda b, thr_ref: (b, 0, 0)),
                pl.BlockSpec((1, 1, S), lambda b, thr_ref: (b, 0, 0)),
                const((1, D)),
                const((1, D)),
                const((1, D)), const((1, D)),
                const((D, 3 * D)), const((1, 3 * D)),
                const((D, D)), const((1, D)),
                const((1, D)), const((1, D)),
                const((D, DFF)), const((1, DFF)),
                const((DFF, D)), const((1, D)),
            ],
            out_specs=pl.BlockSpec((1, S, D), lambda b, thr_ref: (b, 0, 0)),
        ),
        out_shape=jax.ShapeDtypeStruct((B, S, D), jnp.float32),
    )(
        thr,
        x,
        attention_mask.reshape(B, 1, S),
        wp_w.reshape(1, D),
        jnp.broadcast_to(wp_b.reshape(1, 1), (1, D)),
        row(ln1_g), row(ln1_b),
        w_qkv, row(b_qkv),
        w_o, row(b_o),
        row(ln2_g), row(ln2_b),
        w_ff1, row(b_ff1),
        w_ff2, row(b_ff2),
    )
    return out


# speedup vs baseline: 2.7292x; 1.0218x over previous
import jax
import jax.numpy as jnp
import numpy as np
from jax.experimental import pallas as pl
from jax.experimental.pallas import tpu as pltpu

B, S, D = 8, 512, 256
MAX_TOKENS = 512
H, DH, DFF = 8, 32, 1024
K_TOTAL = S * D

ROUTER_CHUNK = 4096
ROUTER_STEPS = K_TOTAL // ROUTER_CHUNK
GRID = ROUTER_STEPS + B


def _fused_kernel(xf_ref, w1_ref, b1_ref, w2_ref, b2_ref,
                  x_ref, mask_ref, wp_ref, wpb_ref,
                  ln1g_ref, ln1b_ref, wqkv_ref, bqkv_ref, wo_ref, bo_ref,
                  ln2g_ref, ln2b_ref, wff1_ref, bff1_ref, wff2_ref, bff2_ref,
                  out_ref, acc_ref, kl_ref):
    i = pl.program_id(0)

    @pl.when(i == 0)
    def _init():
        acc_ref[...] = jnp.zeros_like(acc_ref)

    @pl.when(i < ROUTER_STEPS)
    def _router():
        acc_ref[...] += jnp.dot(
            xf_ref[...].astype(jnp.bfloat16),
            w1_ref[...].astype(jnp.bfloat16),
            preferred_element_type=jnp.float32,
        )

        @pl.when(i == ROUTER_STEPS - 1)
        def _epilogue():
            hdn = acc_ref[...] + b1_ref[...]
            hdn = jnp.where(hdn >= 0, hdn, 0.01 * hdn)
            kl_ref[...] = jnp.sum(hdn * w2_ref[...], axis=1, keepdims=True)

    @pl.when(i >= ROUTER_STEPS)
    def _block_step():
        b = i - ROUTER_STEPS
        xs = x_ref[0]

        kl = kl_ref[pl.ds(b, 1), :] + b2_ref[0, 0]
        thr = jnp.clip(
            jax.nn.sigmoid(kl) * MAX_TOKENS, 1.0, float(MAX_TOKENS)
        ).astype(jnp.int32).astype(jnp.float32)
        thr_f = thr[0, 0]

        weights = jnp.sum(xs * wp_ref[...], axis=1) + wpb_ref[0, 0]
        sel = weights > thr_f
        any_sel = jnp.any(sel)

        @pl.when(jnp.logical_not(any_sel))
        def _copy():
            out_ref[0] = xs

        @pl.when(any_sel)
        def _block():
            def ln(v, g, bb):
                mu = jnp.mean(v, axis=1, keepdims=True)
                var = jnp.mean((v - mu) ** 2, axis=1, keepdims=True)
                return (v - mu) / jnp.sqrt(var + 1e-5) * g + bb

            def mm(a, w):
                return jax.lax.dot_general(
                    a.astype(jnp.bfloat16), w.astype(jnp.bfloat16),
                    (((1,), (0,)), ((), ())),
                    preferred_element_type=jnp.float32,
                )

            a = ln(xs, ln1g_ref[...], ln1b_ref[...])
            qkv = mm(a, wqkv_ref[...]) + bqkv_ref[...]
            bias = mask_ref[0, 0] + jnp.where(sel, 0.0, -1e9)

            ctx_parts = []
            for h in range(H):
                q = qkv[:, h * DH:(h + 1) * DH]
                k = qkv[:, D + h * DH:D + (h + 1) * DH]
                v = qkv[:, 2 * D + h * DH:2 * D + (h + 1) * DH]
                s = jax.lax.dot_general(
                    q.astype(jnp.bfloat16), k.astype(jnp.bfloat16),
                    (((1,), (1,)), ((), ())),
                    preferred_element_type=jnp.float32,
                ) * (1.0 / np.sqrt(DH)) + bias[None, :]
                m = jnp.max(s, axis=1, keepdims=True)
                p = jnp.exp(s - m)
                p = p / jnp.sum(p, axis=1, keepdims=True)
                ctx_parts.append(mm(p, v))
            ctx = jnp.concatenate(ctx_parts, axis=1)

            h1 = xs + mm(ctx, wo_ref[...]) + bo_ref[...]
            m2 = ln(h1, ln2g_ref[...], ln2b_ref[...])
            ff = jax.nn.gelu(mm(m2, wff1_ref[...]) + bff1_ref[...])
            blk = h1 + mm(ff, wff2_ref[...]) + bff2_ref[...]

            selw = jnp.where(sel, weights, 0.0)
            out_ref[0] = xs + selw[:, None] * blk


def kernel(x, attention_mask, wp_w, wp_b, kp_w1, kp_b1, kp_w2, kp_b2,
           ln1_g, ln1_b, w_qkv, b_qkv, w_o, b_o, ln2_g, ln2_b,
           w_ff1, b_ff1, w_ff2, b_ff2):
    x_flat = x.reshape(B, K_TOTAL)

    def rstep(i):
        return jnp.minimum(i, ROUTER_STEPS - 1)

    def bstep(i):
        return jnp.clip(i - ROUTER_STEPS, 0, B - 1)

    row = lambda v: v.reshape(1, -1)
    const = lambda shape: pl.BlockSpec(shape, lambda i: tuple(0 for _ in shape))

    out = pl.pallas_call(
        _fused_kernel,
        grid=(GRID,),
        in_specs=[
            pl.BlockSpec((B, ROUTER_CHUNK), lambda i: (0, rstep(i))),
            pl.BlockSpec((ROUTER_CHUNK, 512), lambda i: (rstep(i), 0)),
            const((1, 512)),
            const((1, 512)),
            const((1, 128)),
            pl.BlockSpec((1, S, D), lambda i: (bstep(i), 0, 0)),
            pl.BlockSpec((1, 1, S), lambda i: (bstep(i), 0, 0)),
            const((1, D)),
            const((1, 128)),
            const((1, D)), const((1, D)),
            const((D, 3 * D)), const((1, 3 * D)),
            const((D, D)), const((1, D)),
            const((1, D)), const((1, D)),
            const((D, DFF)), const((1, DFF)),
            const((DFF, D)), const((1, D)),
        ],
        out_specs=pl.BlockSpec((1, S, D), lambda i: (bstep(i), 0, 0)),
        out_shape=jax.ShapeDtypeStruct((B, S, D), jnp.float32),
        scratch_shapes=[
            pltpu.VMEM((B, 512), jnp.float32),
            pltpu.VMEM((B, 1), jnp.float32),
        ],
    )(
        x_flat, kp_w1, kp_b1.reshape(1, 512), kp_w2.reshape(1, 512),
        jnp.broadcast_to(kp_b2.reshape(1, 1), (1, 128)),
        x,
        attention_mask.reshape(B, 1, S),
        wp_w.reshape(1, D),
        jnp.broadcast_to(wp_b.reshape(1, 1), (1, 128)),
        row(ln1_g), row(ln1_b),
        w_qkv, row(b_qkv),
        w_o, row(b_o),
        row(ln2_g), row(ln2_b),
        w_ff1, row(b_ff1),
        w_ff2, row(b_ff2),
    )
    return out
